# baseline (speedup 1.0000x reference)
import math

import jax
import jax.numpy as jnp
from jax.experimental import pallas as pl

B = 32
N = 256
D_X = 50
D_T = 14
D_OUT = 64
HALF = D_T // 2
_FREQ_SCALE = -(math.log(10000.0) / (HALF - 1))

BB = 32


def _fwd_kernel(t_ref, x_ref, w1_ref, b1_ref, o_ref):
    t = t_ref[...].astype(jnp.float32)
    i = jax.lax.broadcasted_iota(jnp.int32, (1, HALF), 1).astype(jnp.float32)
    freqs = jnp.exp(i * _FREQ_SCALE)
    args = t * freqs
    te = jnp.concatenate([jnp.sin(args), jnp.cos(args)], axis=-1)

    w1 = w1_ref[...]
    wt = w1[:, D_X:]
    cb = (
        jax.lax.dot_general(
            te, wt, (((1,), (1,)), ((), ())),
            preferred_element_type=jnp.float32,
        )
        + b1_ref[...]
    )

    x = x_ref[...]
    y = jax.lax.dot_general(
        x.reshape(BB * N, D_OUT), w1, (((1,), (1,)), ((), ())),
        preferred_element_type=jnp.float32,
    )
    o_ref[...] = y.reshape(BB, N, D_OUT) + cb[:, None, :]


def kernel(x, t, obj_cond, edge_cond_in, relation_cond_in, W1, b1):
    return pl.pallas_call(
        _fwd_kernel,
        grid=(B // BB,),
        in_specs=[
            pl.BlockSpec((BB, 1), lambda b: (b, 0)),
            pl.BlockSpec((BB, N, D_OUT), lambda b: (b, 0, 0)),
            pl.BlockSpec((D_OUT, D_X + D_T), lambda b: (0, 0)),
            pl.BlockSpec((1, D_OUT), lambda b: (0, 0)),
        ],
        out_specs=pl.BlockSpec((BB, N, D_OUT), lambda b: (b, 0, 0)),
        out_shape=jax.ShapeDtypeStruct((B, N, D_OUT), jnp.float32),
    )(t[:, None], jnp.pad(x, ((0, 0), (0, 0), (0, D_OUT - D_X))), W1, b1[None, :])


# speedup vs baseline: 3.0266x; 3.0266x over previous
import math

import jax
import jax.numpy as jnp
from jax.experimental import pallas as pl
from jax.experimental.pallas import tpu as pltpu

B = 32
N = 256
D_X = 50
D_T = 14
D_OUT = 64
HALF = D_T // 2
_FREQ_SCALE = -(math.log(10000.0) / (HALF - 1))

BB = 8


def _fwd_kernel(t_ref, x_ref, w1_ref, b1_ref, o_ref):
    step = pl.program_id(0)
    w1 = w1_ref[...]
    wx = w1[:, :D_X]
    wt = w1[:, D_X:]
    b1_col = jnp.transpose(b1_ref[...], (1, 0))
    i = jax.lax.broadcasted_iota(jnp.int32, (HALF, 1), 0).astype(jnp.float32)
    freqs = jnp.exp(i * _FREQ_SCALE)
    for j in range(BB):
        tb = t_ref[step * BB + j].astype(jnp.float32)
        args = tb * freqs
        te = jnp.concatenate([jnp.sin(args), jnp.cos(args)], axis=0)
        cc = (
            jax.lax.dot_general(
                wt, te, (((1,), (0,)), ((), ())),
                preferred_element_type=jnp.float32,
            )
            + b1_col
        )
        y = jax.lax.dot_general(
            wx, x_ref[:, j, :], (((1,), (0,)), ((), ())),
            preferred_element_type=jnp.float32,
        )
        o_ref[j, :, :] = y + cc


def kernel(x, t, obj_cond, edge_cond_in, relation_cond_in, W1, b1):
    xT = jnp.transpose(x, (2, 0, 1))
    outp = pl.pallas_call(
        _fwd_kernel,
        grid=(B // BB,),
        in_specs=[
            pl.BlockSpec(memory_space=pltpu.SMEM),
            pl.BlockSpec((D_X, BB, N), lambda b: (0, b, 0)),
            pl.BlockSpec((D_OUT, D_X + D_T), lambda b: (0, 0)),
            pl.BlockSpec((1, D_OUT), lambda b: (0, 0)),
        ],
        out_specs=pl.BlockSpec((BB, D_OUT, N), lambda b: (b, 0, 0)),
        out_shape=jax.ShapeDtypeStruct((B, D_OUT, N), jnp.float32),
    )(t, xT, W1, b1[None, :])
    return jnp.transpose(outp, (0, 2, 1))
